# Initial kernel scaffold; baseline (speedup 1.0000x reference)
import jax
import jax.numpy as jnp
from jax.experimental import pallas as pl


def kernel(x, edge_index, batch, W1, b1, W2, b2, Wl, bl):
    raise NotImplementedError("write your pallas kernel here")


# speedup vs baseline: 5.2520x; 5.2520x over previous
import functools

import jax
import jax.numpy as jnp
from jax import lax
from jax.experimental import pallas as pl
from jax.experimental.pallas import tpu as pltpu
from jax.experimental.pallas import tpu_sc as plsc

N = 10000
E = 160000
F_IN = 256
H = 256
OUT = 128
G = 64

NPAD = 10240
EPAD = 163840
DUMMY = N
NC, NS = 2, 16
HALF = 128
RB = 512
NBR = NPAD // RB
ECH = 64
ROWS_T = NPAD // NS
A_CH = EPAD // (NC * NS) // ECH
B_CH = EPAD // NS // ECH

_mesh = plsc.VectorSubcoreMesh(
    core_axis_name="c", subcore_axis_name="s", num_cores=NC, num_subcores=NS)


@functools.partial(
    pl.kernel,
    out_type=jax.ShapeDtypeStruct((NC * NPAD, HALF), jnp.float32),
    mesh=_mesh,
    scratch_types=[
        pltpu.VMEM_SHARED((NPAD, HALF), jnp.float32),
        pltpu.VMEM((ECH,), jnp.int32),
        pltpu.VMEM((ECH, HALF), jnp.float32),
    ],
)
def _deg_kernel(dst_hbm, zeros_hbm, ones_hbm, out_hbm, acc, didx, ones_v):
    c = lax.axis_index("c")
    s = lax.axis_index("s")
    w = c * NS + s
    pltpu.sync_copy(zeros_hbm, acc.at[pl.ds(s * ROWS_T, ROWS_T)])
    pltpu.sync_copy(ones_hbm, ones_v)
    plsc.subcore_barrier()
    dbase = w * A_CH * ECH

    @pl.loop(0, A_CH)
    def _(g):
        pltpu.sync_copy(dst_hbm.at[pl.ds(dbase + g * ECH, ECH)], didx)
        pltpu.sync_copy(ones_v, acc.at[didx], add=True)

    plsc.subcore_barrier()
    pltpu.sync_copy(acc.at[pl.ds(s * ROWS_T, ROWS_T)],
                    out_hbm.at[pl.ds(c * NPAD + s * ROWS_T, ROWS_T)])


@functools.partial(
    pl.kernel,
    out_type=jax.ShapeDtypeStruct((NC * NPAD, HALF), jnp.float32),
    mesh=_mesh,
    scratch_types=[
        pltpu.VMEM_SHARED((NPAD, HALF), jnp.float32),
        pltpu.VMEM((ECH,), jnp.int32),
        pltpu.VMEM((ECH,), jnp.int32),
        pltpu.VMEM((ECH,), jnp.int32),
        pltpu.VMEM((ECH,), jnp.int32),
        pltpu.VMEM((ECH, HALF), jnp.float32),
        pltpu.VMEM((ECH, HALF), jnp.float32),
        pltpu.SemaphoreType.DMA,
        pltpu.SemaphoreType.DMA,
    ],
)
def _prop_kernel(h_hbm, src_hbm, dst_hbm, out_hbm,
                 acc, si0, si1, di0, di1, r0, r1, sem0, sem1):
    c = lax.axis_index("c")
    s = lax.axis_index("s")
    pltpu.sync_copy(h_hbm.at[pl.ds(c * NPAD + s * ROWS_T, ROWS_T)],
                    acc.at[pl.ds(s * ROWS_T, ROWS_T)])
    plsc.subcore_barrier()

    sbase = (c * NS + s) * B_CH * ECH
    dbase = s * B_CH * ECH

    @pl.loop(0, B_CH, step=2)
    def _(g):
        pltpu.sync_copy(src_hbm.at[pl.ds(sbase + g * ECH, ECH)], si0)
        pltpu.sync_copy(src_hbm.at[pl.ds(sbase + (g + 1) * ECH, ECH)], si1)
        pltpu.sync_copy(dst_hbm.at[pl.ds(dbase + g * ECH, ECH)], di0)
        pltpu.sync_copy(dst_hbm.at[pl.ds(dbase + (g + 1) * ECH, ECH)], di1)
        cp0 = pltpu.async_copy(h_hbm.at[si0], r0, sem0)
        cp1 = pltpu.async_copy(h_hbm.at[si1], r1, sem1)
        cp0.wait()
        pltpu.sync_copy(r0, acc.at[di0], add=True)
        cp1.wait()
        pltpu.sync_copy(r1, acc.at[di1], add=True)

    plsc.subcore_barrier()
    pltpu.sync_copy(acc.at[pl.ds(s * ROWS_T, ROWS_T)],
                    out_hbm.at[pl.ds(c * NPAD + s * ROWS_T, ROWS_T)])


def _dinv_block(deg0, deg1):
    deg = deg0[:, :1] + deg1[:, :1] + 1.0
    return lax.rsqrt(jnp.maximum(deg, 1.0))


def _hp1_body(x_ref, w_ref, deg0_ref, deg1_ref, out_ref):
    dinv = _dinv_block(deg0_ref[...], deg1_ref[...])
    h = jnp.dot(x_ref[...], w_ref[...], preferred_element_type=jnp.float32)
    out_ref[...] = dinv * h


def _hp2_body(a0_ref, a1_ref, deg0_ref, deg1_ref, b_ref, w2a_ref, w2b_ref,
              out_ref):
    dinv = _dinv_block(deg0_ref[...], deg1_ref[...])
    b = b_ref[...]
    r0 = jnp.maximum(dinv * a0_ref[...] + b[0:1, :], 0.0)
    r1 = jnp.maximum(dinv * a1_ref[...] + b[1:2, :], 0.0)
    h = (jnp.dot(r0, w2a_ref[...], preferred_element_type=jnp.float32)
         + jnp.dot(r1, w2b_ref[...], preferred_element_type=jnp.float32))
    out_ref[...] = dinv * h


def _final_body(a0_ref, a1_ref, deg0_ref, deg1_ref, b_ref, batch_ref,
                wl_ref, bl_ref, out_ref, pooled, cnt):
    ib = pl.program_id(0)

    @pl.when(ib == 0)
    def _():
        pooled[...] = jnp.zeros_like(pooled)
        cnt[...] = jnp.zeros_like(cnt)

    dinv = _dinv_block(deg0_ref[...], deg1_ref[...])
    b = b_ref[...]
    r0 = jnp.maximum(dinv * a0_ref[...] + b[0:1, :], 0.0)
    r1 = jnp.maximum(dinv * a1_ref[...] + b[1:2, :], 0.0)
    h = jnp.concatenate([r0, r1], axis=1)
    bt = batch_ref[...][0]
    m = (lax.broadcasted_iota(jnp.int32, (G, RB), 0) == bt
         ).astype(jnp.float32)
    pooled[...] += jnp.dot(m, h, preferred_element_type=jnp.float32)
    cnt[...] += jnp.broadcast_to(jnp.sum(m, axis=1, keepdims=True), (G, 128))

    @pl.when(ib == pl.num_programs(0) - 1)
    def _():
        p = pooled[...] / jnp.maximum(cnt[:, :1], 1.0)
        out_ref[...] = (jnp.dot(p, wl_ref[...],
                                preferred_element_type=jnp.float32)
                        + bl_ref[...])


def _row_spec(cols):
    return pl.BlockSpec((RB, cols), lambda c, ib: (ib, 0))


def _hp1_call(x, w1, deg0, deg1):
    return pl.pallas_call(
        _hp1_body,
        grid=(NC, NBR),
        in_specs=[
            _row_spec(F_IN),
            pl.BlockSpec((F_IN, HALF), lambda c, ib: (0, c)),
            _row_spec(HALF), _row_spec(HALF),
        ],
        out_specs=pl.BlockSpec((RB, HALF), lambda c, ib: (c * NBR + ib, 0)),
        out_shape=jax.ShapeDtypeStruct((NC * NPAD, HALF), jnp.float32),
    )(x, w1, deg0, deg1)


def _hp2_call(a0, a1, deg0, deg1, b1r, w2):
    return pl.pallas_call(
        _hp2_body,
        grid=(NC, NBR),
        in_specs=[
            _row_spec(HALF), _row_spec(HALF),
            _row_spec(HALF), _row_spec(HALF),
            pl.BlockSpec((2, HALF), lambda c, ib: (0, 0)),
            pl.BlockSpec((HALF, HALF), lambda c, ib: (0, c)),
            pl.BlockSpec((HALF, HALF), lambda c, ib: (1, c)),
        ],
        out_specs=pl.BlockSpec((RB, HALF), lambda c, ib: (c * NBR + ib, 0)),
        out_shape=jax.ShapeDtypeStruct((NC * NPAD, HALF), jnp.float32),
    )(a0, a1, deg0, deg1, b1r, w2, w2)


def _final_call(a0, a1, deg0, deg1, b2r, batch3d, wl, bl2d):
    spec1 = pl.BlockSpec((RB, HALF), lambda ib: (ib, 0))
    spec16 = pl.BlockSpec((RB, HALF), lambda ib: (ib, 0))
    return pl.pallas_call(
        _final_body,
        grid=(NBR,),
        in_specs=[
            spec1, spec1, spec16, spec16,
            pl.BlockSpec((2, HALF), lambda ib: (0, 0)),
            pl.BlockSpec((1, 1, RB), lambda ib: (ib, 0, 0)),
            pl.BlockSpec((H, OUT), lambda ib: (0, 0)),
            pl.BlockSpec((1, OUT), lambda ib: (0, 0)),
        ],
        out_specs=pl.BlockSpec((G, OUT), lambda ib: (0, 0)),
        out_shape=jax.ShapeDtypeStruct((G, OUT), jnp.float32),
        scratch_shapes=[
            pltpu.VMEM((G, H), jnp.float32),
            pltpu.VMEM((G, 128), jnp.float32),
        ],
    )(a0, a1, deg0, deg1, b2r, batch3d, wl, bl2d)


def kernel(x, edge_index, batch, W1, b1, W2, b2, Wl, bl):
    src = edge_index[0].astype(jnp.int32)
    dst = edge_index[1].astype(jnp.int32)
    pad_e = EPAD - E
    srcp = jnp.concatenate([src, jnp.full((pad_e,), DUMMY, jnp.int32)])
    dstp = jnp.concatenate([dst, jnp.full((pad_e,), DUMMY, jnp.int32)])
    srcA = jnp.concatenate([srcp, srcp + NPAD])
    xp = jnp.zeros((NPAD, F_IN), jnp.float32).at[:N].set(x)
    batchp = jnp.full((NPAD,), G, jnp.int32).at[:N].set(batch.astype(jnp.int32))
    batch3d = batchp.reshape(NBR, 1, RB)
    zeros_h = jnp.zeros((ROWS_T, HALF), jnp.float32)
    ones_h = jnp.ones((ECH, HALF), jnp.float32)
    b1r = b1.reshape(2, HALF)
    b2r = b2.reshape(2, HALF)
    bl2d = bl.reshape(1, OUT)

    deg = _deg_kernel(dstp, zeros_h, ones_h)
    deg0, deg1 = deg[:NPAD], deg[NPAD:]

    hp1 = _hp1_call(xp, W1, deg0, deg1)
    a1 = _prop_kernel(hp1, srcA, dstp)
    hp2 = _hp2_call(a1[:NPAD], a1[NPAD:], deg0, deg1, b1r, W2)
    a2 = _prop_kernel(hp2, srcA, dstp)
    return _final_call(a2[:NPAD], a2[NPAD:], deg0, deg1, b2r, batch3d, Wl,
                       bl2d)


# speedup vs baseline: 6.1345x; 1.1680x over previous
import functools

import jax
import jax.numpy as jnp
from jax import lax
from jax.experimental import pallas as pl
from jax.experimental.pallas import tpu as pltpu
from jax.experimental.pallas import tpu_sc as plsc

N = 10000
E = 160000
F_IN = 256
H = 256
OUT = 128
G = 64

NPAD = 10240
EPAD = 163840
DUMMY = N
NC, NS = 2, 16
HALF = 128
RB = 512
NBR = NPAD // RB
ECH = 128
ROWS_T = NPAD // NS
A_CH = EPAD // (NC * NS) // ECH
B_CH = EPAD // NS // ECH

_mesh = plsc.VectorSubcoreMesh(
    core_axis_name="c", subcore_axis_name="s", num_cores=NC, num_subcores=NS)


@functools.partial(
    pl.kernel,
    out_type=jax.ShapeDtypeStruct((NC * NPAD, HALF), jnp.float32),
    mesh=_mesh,
    scratch_types=[
        pltpu.VMEM_SHARED((NPAD, HALF), jnp.float32),
        pltpu.VMEM((ECH,), jnp.int32),
        pltpu.VMEM((ECH, HALF), jnp.float32),
    ],
)
def _deg_kernel(dst_hbm, zeros_hbm, ones_hbm, out_hbm, acc, didx, ones_v):
    c = lax.axis_index("c")
    s = lax.axis_index("s")
    w = c * NS + s
    pltpu.sync_copy(zeros_hbm, acc.at[pl.ds(s * ROWS_T, ROWS_T)])
    pltpu.sync_copy(ones_hbm, ones_v)
    plsc.subcore_barrier()
    dbase = w * A_CH * ECH

    @pl.loop(0, A_CH)
    def _(g):
        pltpu.sync_copy(dst_hbm.at[pl.ds(dbase + g * ECH, ECH)], didx)
        pltpu.sync_copy(ones_v, acc.at[didx], add=True)

    plsc.subcore_barrier()
    pltpu.sync_copy(acc.at[pl.ds(s * ROWS_T, ROWS_T)],
                    out_hbm.at[pl.ds(c * NPAD + s * ROWS_T, ROWS_T)])


@functools.partial(
    pl.kernel,
    out_type=jax.ShapeDtypeStruct((NC * NPAD, HALF), jnp.float32),
    mesh=_mesh,
    scratch_types=[
        pltpu.VMEM_SHARED((NPAD, HALF), jnp.float32),
        pltpu.VMEM((ECH,), jnp.int32),
        pltpu.VMEM((ECH,), jnp.int32),
        pltpu.VMEM((ECH,), jnp.int32),
        pltpu.VMEM((ECH,), jnp.int32),
        pltpu.VMEM((ECH, HALF), jnp.float32),
        pltpu.VMEM((ECH, HALF), jnp.float32),
        pltpu.SemaphoreType.DMA,
        pltpu.SemaphoreType.DMA,
    ],
)
def _prop_kernel(h_hbm, src_hbm, dst_hbm, out_hbm,
                 acc, si0, si1, di0, di1, r0, r1, sem0, sem1):
    c = lax.axis_index("c")
    s = lax.axis_index("s")
    pltpu.sync_copy(h_hbm.at[pl.ds(c * NPAD + s * ROWS_T, ROWS_T)],
                    acc.at[pl.ds(s * ROWS_T, ROWS_T)])
    plsc.subcore_barrier()

    sbase = (c * NS + s) * B_CH * ECH
    dbase = s * B_CH * ECH

    @pl.loop(0, B_CH, step=2)
    def _(g):
        pltpu.sync_copy(src_hbm.at[pl.ds(sbase + g * ECH, ECH)], si0)
        pltpu.sync_copy(src_hbm.at[pl.ds(sbase + (g + 1) * ECH, ECH)], si1)
        pltpu.sync_copy(dst_hbm.at[pl.ds(dbase + g * ECH, ECH)], di0)
        pltpu.sync_copy(dst_hbm.at[pl.ds(dbase + (g + 1) * ECH, ECH)], di1)
        cp0 = pltpu.async_copy(h_hbm.at[si0], r0, sem0)
        cp1 = pltpu.async_copy(h_hbm.at[si1], r1, sem1)
        cp0.wait()
        pltpu.sync_copy(r0, acc.at[di0], add=True)
        cp1.wait()
        pltpu.sync_copy(r1, acc.at[di1], add=True)

    plsc.subcore_barrier()
    pltpu.sync_copy(acc.at[pl.ds(s * ROWS_T, ROWS_T)],
                    out_hbm.at[pl.ds(c * NPAD + s * ROWS_T, ROWS_T)])


def _dinv_block(deg0, deg1):
    deg = deg0[:, :1] + deg1[:, :1] + 1.0
    return lax.rsqrt(jnp.maximum(deg, 1.0))


def _hp1_body(x_ref, w_ref, deg0_ref, deg1_ref, out_ref):
    dinv = _dinv_block(deg0_ref[...], deg1_ref[...])
    h = jnp.dot(x_ref[...], w_ref[...], preferred_element_type=jnp.float32)
    out_ref[...] = dinv * h


def _hp2_body(a0_ref, a1_ref, deg0_ref, deg1_ref, b_ref, w2a_ref, w2b_ref,
              out_ref):
    dinv = _dinv_block(deg0_ref[...], deg1_ref[...])
    b = b_ref[...]
    r0 = jnp.maximum(dinv * a0_ref[...] + b[0:1, :], 0.0)
    r1 = jnp.maximum(dinv * a1_ref[...] + b[1:2, :], 0.0)
    h = (jnp.dot(r0, w2a_ref[...], preferred_element_type=jnp.float32)
         + jnp.dot(r1, w2b_ref[...], preferred_element_type=jnp.float32))
    out_ref[...] = dinv * h


def _final_body(a0_ref, a1_ref, deg0_ref, deg1_ref, b_ref, batch_ref,
                wl_ref, bl_ref, out_ref, pooled, cnt):
    ib = pl.program_id(0)

    @pl.when(ib == 0)
    def _():
        pooled[...] = jnp.zeros_like(pooled)
        cnt[...] = jnp.zeros_like(cnt)

    dinv = _dinv_block(deg0_ref[...], deg1_ref[...])
    b = b_ref[...]
    r0 = jnp.maximum(dinv * a0_ref[...] + b[0:1, :], 0.0)
    r1 = jnp.maximum(dinv * a1_ref[...] + b[1:2, :], 0.0)
    h = jnp.concatenate([r0, r1], axis=1)
    bt = batch_ref[...][0]
    m = (lax.broadcasted_iota(jnp.int32, (G, RB), 0) == bt
         ).astype(jnp.float32)
    pooled[...] += jnp.dot(m, h, preferred_element_type=jnp.float32)
    cnt[...] += jnp.broadcast_to(jnp.sum(m, axis=1, keepdims=True), (G, 128))

    @pl.when(ib == pl.num_programs(0) - 1)
    def _():
        p = pooled[...] / jnp.maximum(cnt[:, :1], 1.0)
        out_ref[...] = (jnp.dot(p, wl_ref[...],
                                preferred_element_type=jnp.float32)
                        + bl_ref[...])


def _row_spec(cols):
    return pl.BlockSpec((RB, cols), lambda c, ib: (ib, 0))


def _hp1_call(x, w1, deg0, deg1):
    return pl.pallas_call(
        _hp1_body,
        grid=(NC, NBR),
        in_specs=[
            _row_spec(F_IN),
            pl.BlockSpec((F_IN, HALF), lambda c, ib: (0, c)),
            _row_spec(HALF), _row_spec(HALF),
        ],
        out_specs=pl.BlockSpec((RB, HALF), lambda c, ib: (c * NBR + ib, 0)),
        out_shape=jax.ShapeDtypeStruct((NC * NPAD, HALF), jnp.float32),
    )(x, w1, deg0, deg1)


def _hp2_call(a0, a1, deg0, deg1, b1r, w2):
    return pl.pallas_call(
        _hp2_body,
        grid=(NC, NBR),
        in_specs=[
            _row_spec(HALF), _row_spec(HALF),
            _row_spec(HALF), _row_spec(HALF),
            pl.BlockSpec((2, HALF), lambda c, ib: (0, 0)),
            pl.BlockSpec((HALF, HALF), lambda c, ib: (0, c)),
            pl.BlockSpec((HALF, HALF), lambda c, ib: (1, c)),
        ],
        out_specs=pl.BlockSpec((RB, HALF), lambda c, ib: (c * NBR + ib, 0)),
        out_shape=jax.ShapeDtypeStruct((NC * NPAD, HALF), jnp.float32),
    )(a0, a1, deg0, deg1, b1r, w2, w2)


def _final_call(a0, a1, deg0, deg1, b2r, batch3d, wl, bl2d):
    spec1 = pl.BlockSpec((RB, HALF), lambda ib: (ib, 0))
    spec16 = pl.BlockSpec((RB, HALF), lambda ib: (ib, 0))
    return pl.pallas_call(
        _final_body,
        grid=(NBR,),
        in_specs=[
            spec1, spec1, spec16, spec16,
            pl.BlockSpec((2, HALF), lambda ib: (0, 0)),
            pl.BlockSpec((1, 1, RB), lambda ib: (ib, 0, 0)),
            pl.BlockSpec((H, OUT), lambda ib: (0, 0)),
            pl.BlockSpec((1, OUT), lambda ib: (0, 0)),
        ],
        out_specs=pl.BlockSpec((G, OUT), lambda ib: (0, 0)),
        out_shape=jax.ShapeDtypeStruct((G, OUT), jnp.float32),
        scratch_shapes=[
            pltpu.VMEM((G, H), jnp.float32),
            pltpu.VMEM((G, 128), jnp.float32),
        ],
    )(a0, a1, deg0, deg1, b2r, batch3d, wl, bl2d)


def kernel(x, edge_index, batch, W1, b1, W2, b2, Wl, bl):
    src = edge_index[0].astype(jnp.int32)
    dst = edge_index[1].astype(jnp.int32)
    pad_e = EPAD - E
    srcp = jnp.concatenate([src, jnp.full((pad_e,), DUMMY, jnp.int32)])
    dstp = jnp.concatenate([dst, jnp.full((pad_e,), DUMMY, jnp.int32)])
    srcA = jnp.concatenate([srcp, srcp + NPAD])
    xp = jnp.zeros((NPAD, F_IN), jnp.float32).at[:N].set(x)
    batchp = jnp.full((NPAD,), G, jnp.int32).at[:N].set(batch.astype(jnp.int32))
    batch3d = batchp.reshape(NBR, 1, RB)
    zeros_h = jnp.zeros((ROWS_T, HALF), jnp.float32)
    ones_h = jnp.ones((ECH, HALF), jnp.float32)
    b1r = b1.reshape(2, HALF)
    b2r = b2.reshape(2, HALF)
    bl2d = bl.reshape(1, OUT)

    deg = _deg_kernel(dstp, zeros_h, ones_h)
    deg0, deg1 = deg[:NPAD], deg[NPAD:]

    hp1 = _hp1_call(xp, W1, deg0, deg1)
    a1 = _prop_kernel(hp1, srcA, dstp)
    hp2 = _hp2_call(a1[:NPAD], a1[NPAD:], deg0, deg1, b1r, W2)
    a2 = _prop_kernel(hp2, srcA, dstp)
    return _final_call(a2[:NPAD], a2[NPAD:], deg0, deg1, b2r, batch3d, Wl,
                       bl2d)


# speedup vs baseline: 7.0133x; 1.1433x over previous
import functools

import jax
import jax.numpy as jnp
from jax import lax
from jax.experimental import pallas as pl
from jax.experimental.pallas import tpu as pltpu
from jax.experimental.pallas import tpu_sc as plsc

N = 10000
E = 160000
F_IN = 256
H = 256
OUT = 128
G = 64

NPAD = 10240
EPAD = 163840
DUMMY = N
NC, NS = 2, 16
HALF = 128
RB = 512
NBR = NPAD // RB
ECH = 128
ROWS_T = NPAD // NS
A_CH = EPAD // (NC * NS) // ECH
B_CH = EPAD // NS // ECH

_mesh = plsc.VectorSubcoreMesh(
    core_axis_name="c", subcore_axis_name="s", num_cores=NC, num_subcores=NS)


@functools.partial(
    pl.kernel,
    out_type=jax.ShapeDtypeStruct((NC * NPAD, HALF), jnp.float32),
    mesh=_mesh,
    scratch_types=[
        pltpu.VMEM_SHARED((NPAD, HALF), jnp.float32),
        pltpu.VMEM((ECH,), jnp.int32),
        pltpu.VMEM((ECH, HALF), jnp.float32),
    ],
)
def _deg_kernel(dst_hbm, zeros_hbm, ones_hbm, out_hbm, acc, didx, ones_v):
    c = lax.axis_index("c")
    s = lax.axis_index("s")
    w = c * NS + s
    pltpu.sync_copy(zeros_hbm, acc.at[pl.ds(s * ROWS_T, ROWS_T)])
    pltpu.sync_copy(ones_hbm, ones_v)
    plsc.subcore_barrier()
    dbase = w * A_CH * ECH

    @pl.loop(0, A_CH)
    def _(g):
        pltpu.sync_copy(dst_hbm.at[pl.ds(dbase + g * ECH, ECH)], didx)
        pltpu.sync_copy(ones_v, acc.at[didx], add=True)

    plsc.subcore_barrier()
    pltpu.sync_copy(acc.at[pl.ds(s * ROWS_T, ROWS_T)],
                    out_hbm.at[pl.ds(c * NPAD + s * ROWS_T, ROWS_T)])


@functools.partial(
    pl.kernel,
    out_type=jax.ShapeDtypeStruct((NC * NPAD, HALF), jnp.float32),
    mesh=_mesh,
    scratch_types=[
        pltpu.VMEM_SHARED((NPAD, HALF), jnp.float32),
        pltpu.VMEM((4, ECH), jnp.int32),
        pltpu.VMEM((4, ECH), jnp.int32),
        pltpu.VMEM((ECH, HALF), jnp.float32),
        pltpu.VMEM((ECH, HALF), jnp.float32),
        pltpu.SemaphoreType.DMA,
        pltpu.SemaphoreType.DMA,
    ],
)
def _prop_kernel(h_hbm, src_hbm, dst_hbm, out_hbm,
                 acc, si, di, r0, r1, sem0, sem1):
    c = lax.axis_index("c")
    s = lax.axis_index("s")
    pltpu.sync_copy(h_hbm.at[pl.ds(c * NPAD + s * ROWS_T, ROWS_T)],
                    acc.at[pl.ds(s * ROWS_T, ROWS_T)])
    plsc.subcore_barrier()

    srow = (c * NS + s) * B_CH
    drow = s * B_CH

    @pl.loop(0, B_CH, step=4)
    def _(g):
        pltpu.sync_copy(src_hbm.at[pl.ds(srow + g, 4)], si)
        pltpu.sync_copy(dst_hbm.at[pl.ds(drow + g, 4)], di)
        cp0 = pltpu.async_copy(h_hbm.at[si.at[0]], r0, sem0)
        cp1 = pltpu.async_copy(h_hbm.at[si.at[1]], r1, sem1)
        cp0.wait()
        pltpu.sync_copy(r0, acc.at[di.at[0]], add=True)
        cp2 = pltpu.async_copy(h_hbm.at[si.at[2]], r0, sem0)
        cp1.wait()
        pltpu.sync_copy(r1, acc.at[di.at[1]], add=True)
        cp3 = pltpu.async_copy(h_hbm.at[si.at[3]], r1, sem1)
        cp2.wait()
        pltpu.sync_copy(r0, acc.at[di.at[2]], add=True)
        cp3.wait()
        pltpu.sync_copy(r1, acc.at[di.at[3]], add=True)

    plsc.subcore_barrier()
    pltpu.sync_copy(acc.at[pl.ds(s * ROWS_T, ROWS_T)],
                    out_hbm.at[pl.ds(c * NPAD + s * ROWS_T, ROWS_T)])


def _dinv_block(deg0, deg1):
    deg = deg0[:, :1] + deg1[:, :1] + 1.0
    return lax.rsqrt(jnp.maximum(deg, 1.0))


def _hp1_body(x_ref, w_ref, deg0_ref, deg1_ref, out_ref):
    dinv = _dinv_block(deg0_ref[...], deg1_ref[...])
    h = jnp.dot(x_ref[...], w_ref[...], preferred_element_type=jnp.float32)
    out_ref[...] = dinv * h


def _hp2_body(a0_ref, a1_ref, deg0_ref, deg1_ref, b_ref, w2a_ref, w2b_ref,
              out_ref):
    dinv = _dinv_block(deg0_ref[...], deg1_ref[...])
    b = b_ref[...]
    r0 = jnp.maximum(dinv * a0_ref[...] + b[0:1, :], 0.0)
    r1 = jnp.maximum(dinv * a1_ref[...] + b[1:2, :], 0.0)
    h = (jnp.dot(r0, w2a_ref[...], preferred_element_type=jnp.float32)
         + jnp.dot(r1, w2b_ref[...], preferred_element_type=jnp.float32))
    out_ref[...] = dinv * h


def _final_body(a0_ref, a1_ref, deg0_ref, deg1_ref, b_ref, batch_ref,
                wl_ref, bl_ref, out_ref, pooled, cnt):
    ib = pl.program_id(0)

    @pl.when(ib == 0)
    def _():
        pooled[...] = jnp.zeros_like(pooled)
        cnt[...] = jnp.zeros_like(cnt)

    dinv = _dinv_block(deg0_ref[...], deg1_ref[...])
    b = b_ref[...]
    r0 = jnp.maximum(dinv * a0_ref[...] + b[0:1, :], 0.0)
    r1 = jnp.maximum(dinv * a1_ref[...] + b[1:2, :], 0.0)
    h = jnp.concatenate([r0, r1], axis=1)
    bt = batch_ref[...][0]
    m = (lax.broadcasted_iota(jnp.int32, (G, RB), 0) == bt
         ).astype(jnp.float32)
    pooled[...] += jnp.dot(m, h, preferred_element_type=jnp.float32)
    cnt[...] += jnp.broadcast_to(jnp.sum(m, axis=1, keepdims=True), (G, 128))

    @pl.when(ib == pl.num_programs(0) - 1)
    def _():
        p = pooled[...] / jnp.maximum(cnt[:, :1], 1.0)
        out_ref[...] = (jnp.dot(p, wl_ref[...],
                                preferred_element_type=jnp.float32)
                        + bl_ref[...])


def _row_spec(cols):
    return pl.BlockSpec((RB, cols), lambda c, ib: (ib, 0))


def _hp1_call(x, w1, deg0, deg1):
    return pl.pallas_call(
        _hp1_body,
        grid=(NC, NBR),
        in_specs=[
            _row_spec(F_IN),
            pl.BlockSpec((F_IN, HALF), lambda c, ib: (0, c)),
            _row_spec(HALF), _row_spec(HALF),
        ],
        out_specs=pl.BlockSpec((RB, HALF), lambda c, ib: (c * NBR + ib, 0)),
        out_shape=jax.ShapeDtypeStruct((NC * NPAD, HALF), jnp.float32),
    )(x, w1, deg0, deg1)


def _hp2_call(a0, a1, deg0, deg1, b1r, w2):
    return pl.pallas_call(
        _hp2_body,
        grid=(NC, NBR),
        in_specs=[
            _row_spec(HALF), _row_spec(HALF),
            _row_spec(HALF), _row_spec(HALF),
            pl.BlockSpec((2, HALF), lambda c, ib: (0, 0)),
            pl.BlockSpec((HALF, HALF), lambda c, ib: (0, c)),
            pl.BlockSpec((HALF, HALF), lambda c, ib: (1, c)),
        ],
        out_specs=pl.BlockSpec((RB, HALF), lambda c, ib: (c * NBR + ib, 0)),
        out_shape=jax.ShapeDtypeStruct((NC * NPAD, HALF), jnp.float32),
    )(a0, a1, deg0, deg1, b1r, w2, w2)


def _final_call(a0, a1, deg0, deg1, b2r, batch3d, wl, bl2d):
    spec1 = pl.BlockSpec((RB, HALF), lambda ib: (ib, 0))
    spec16 = pl.BlockSpec((RB, HALF), lambda ib: (ib, 0))
    return pl.pallas_call(
        _final_body,
        grid=(NBR,),
        in_specs=[
            spec1, spec1, spec16, spec16,
            pl.BlockSpec((2, HALF), lambda ib: (0, 0)),
            pl.BlockSpec((1, 1, RB), lambda ib: (ib, 0, 0)),
            pl.BlockSpec((H, OUT), lambda ib: (0, 0)),
            pl.BlockSpec((1, OUT), lambda ib: (0, 0)),
        ],
        out_specs=pl.BlockSpec((G, OUT), lambda ib: (0, 0)),
        out_shape=jax.ShapeDtypeStruct((G, OUT), jnp.float32),
        scratch_shapes=[
            pltpu.VMEM((G, H), jnp.float32),
            pltpu.VMEM((G, 128), jnp.float32),
        ],
    )(a0, a1, deg0, deg1, b2r, batch3d, wl, bl2d)


def kernel(x, edge_index, batch, W1, b1, W2, b2, Wl, bl):
    src = edge_index[0].astype(jnp.int32)
    dst = edge_index[1].astype(jnp.int32)
    pad_e = EPAD - E
    srcp = jnp.concatenate([src, jnp.full((pad_e,), DUMMY, jnp.int32)])
    dstp = jnp.concatenate([dst, jnp.full((pad_e,), DUMMY, jnp.int32)])
    srcA = jnp.concatenate([srcp, srcp + NPAD]).reshape(-1, ECH)
    dst2d = dstp.reshape(-1, ECH)
    xp = jnp.zeros((NPAD, F_IN), jnp.float32).at[:N].set(x)
    batchp = jnp.full((NPAD,), G, jnp.int32).at[:N].set(batch.astype(jnp.int32))
    batch3d = batchp.reshape(NBR, 1, RB)
    zeros_h = jnp.zeros((ROWS_T, HALF), jnp.float32)
    ones_h = jnp.ones((ECH, HALF), jnp.float32)
    b1r = b1.reshape(2, HALF)
    b2r = b2.reshape(2, HALF)
    bl2d = bl.reshape(1, OUT)

    deg = _deg_kernel(dstp, zeros_h, ones_h)
    deg0, deg1 = deg[:NPAD], deg[NPAD:]

    hp1 = _hp1_call(xp, W1, deg0, deg1)
    a1 = _prop_kernel(hp1, srcA, dst2d)
    hp2 = _hp2_call(a1[:NPAD], a1[NPAD:], deg0, deg1, b1r, W2)
    a2 = _prop_kernel(hp2, srcA, dst2d)
    return _final_call(a2[:NPAD], a2[NPAD:], deg0, deg1, b2r, batch3d, Wl,
                       bl2d)


# speedup vs baseline: 7.1823x; 1.0241x over previous
import functools

import jax
import jax.numpy as jnp
from jax import lax
from jax.experimental import pallas as pl
from jax.experimental.pallas import tpu as pltpu
from jax.experimental.pallas import tpu_sc as plsc

N = 10000
E = 160000
F_IN = 256
H = 256
OUT = 128
G = 64

NPAD = 10240
EPAD = 163840
DUMMY = N
NC, NS = 2, 16
HALF = 128
RB = 512
NBR = NPAD // RB
ECH = 128
ROWS_T = NPAD // NS
A_CH = EPAD // (NC * NS) // ECH
B_CH = EPAD // NS // ECH

_mesh = plsc.VectorSubcoreMesh(
    core_axis_name="c", subcore_axis_name="s", num_cores=NC, num_subcores=NS)


@functools.partial(
    pl.kernel,
    out_type=jax.ShapeDtypeStruct((NC * NPAD, HALF), jnp.float32),
    mesh=_mesh,
    scratch_types=[
        pltpu.VMEM_SHARED((NPAD, HALF), jnp.float32),
        pltpu.VMEM((4, ECH), jnp.int32),
        pltpu.VMEM((ECH, HALF), jnp.float32),
    ],
)
def _deg_kernel(dst_hbm, zeros_hbm, ones_hbm, out_hbm, acc, di, ones_v):
    c = lax.axis_index("c")
    s = lax.axis_index("s")
    w = c * NS + s
    pltpu.sync_copy(zeros_hbm, acc.at[pl.ds(s * ROWS_T, ROWS_T)])
    pltpu.sync_copy(ones_hbm, ones_v)
    plsc.subcore_barrier()
    drow = w * A_CH

    @pl.loop(0, A_CH, step=4)
    def _(g):
        pltpu.sync_copy(dst_hbm.at[pl.ds(drow + g, 4)], di)
        pltpu.sync_copy(ones_v, acc.at[di.at[0]], add=True)
        pltpu.sync_copy(ones_v, acc.at[di.at[1]], add=True)
        pltpu.sync_copy(ones_v, acc.at[di.at[2]], add=True)
        pltpu.sync_copy(ones_v, acc.at[di.at[3]], add=True)

    plsc.subcore_barrier()
    pltpu.sync_copy(acc.at[pl.ds(s * ROWS_T, ROWS_T)],
                    out_hbm.at[pl.ds(c * NPAD + s * ROWS_T, ROWS_T)])


@functools.partial(
    pl.kernel,
    out_type=jax.ShapeDtypeStruct((NC * NPAD, HALF), jnp.float32),
    mesh=_mesh,
    scratch_types=[
        pltpu.VMEM_SHARED((NPAD, HALF), jnp.float32),
        pltpu.VMEM((4, ECH), jnp.int32),
        pltpu.VMEM((4, ECH), jnp.int32),
        pltpu.VMEM((ECH, HALF), jnp.float32),
        pltpu.VMEM((ECH, HALF), jnp.float32),
        pltpu.SemaphoreType.DMA,
        pltpu.SemaphoreType.DMA,
    ],
)
def _prop_kernel(h_hbm, src_hbm, dst_hbm, out_hbm,
                 acc, si, di, r0, r1, sem0, sem1):
    c = lax.axis_index("c")
    s = lax.axis_index("s")
    pltpu.sync_copy(h_hbm.at[pl.ds(c * NPAD + s * ROWS_T, ROWS_T)],
                    acc.at[pl.ds(s * ROWS_T, ROWS_T)])
    plsc.subcore_barrier()

    srow = (c * NS + s) * B_CH
    drow = s * B_CH

    @pl.loop(0, B_CH, step=4)
    def _(g):
        pltpu.sync_copy(src_hbm.at[pl.ds(srow + g, 4)], si)
        pltpu.sync_copy(dst_hbm.at[pl.ds(drow + g, 4)], di)
        cp0 = pltpu.async_copy(h_hbm.at[si.at[0]], r0, sem0)
        cp1 = pltpu.async_copy(h_hbm.at[si.at[1]], r1, sem1)
        cp0.wait()
        pltpu.sync_copy(r0, acc.at[di.at[0]], add=True)
        cp2 = pltpu.async_copy(h_hbm.at[si.at[2]], r0, sem0)
        cp1.wait()
        pltpu.sync_copy(r1, acc.at[di.at[1]], add=True)
        cp3 = pltpu.async_copy(h_hbm.at[si.at[3]], r1, sem1)
        cp2.wait()
        pltpu.sync_copy(r0, acc.at[di.at[2]], add=True)
        cp3.wait()
        pltpu.sync_copy(r1, acc.at[di.at[3]], add=True)

    plsc.subcore_barrier()
    pltpu.sync_copy(acc.at[pl.ds(s * ROWS_T, ROWS_T)],
                    out_hbm.at[pl.ds(c * NPAD + s * ROWS_T, ROWS_T)])


def _dinv_block(deg0, deg1):
    deg = deg0[:, :1] + deg1[:, :1] + 1.0
    return lax.rsqrt(jnp.maximum(deg, 1.0))


def _hp1_body(x_ref, w_ref, deg0_ref, deg1_ref, out_ref):
    dinv = _dinv_block(deg0_ref[...], deg1_ref[...])
    h = jnp.dot(x_ref[...], w_ref[...], preferred_element_type=jnp.float32)
    out_ref[...] = dinv * h


def _hp2_body(a0_ref, a1_ref, deg0_ref, deg1_ref, b_ref, w2a_ref, w2b_ref,
              out_ref):
    dinv = _dinv_block(deg0_ref[...], deg1_ref[...])
    b = b_ref[...]
    r0 = jnp.maximum(dinv * a0_ref[...] + b[0:1, :], 0.0)
    r1 = jnp.maximum(dinv * a1_ref[...] + b[1:2, :], 0.0)
    h = (jnp.dot(r0, w2a_ref[...], preferred_element_type=jnp.float32)
         + jnp.dot(r1, w2b_ref[...], preferred_element_type=jnp.float32))
    out_ref[...] = dinv * h


def _final_body(a0_ref, a1_ref, deg0_ref, deg1_ref, b_ref, batch_ref,
                wl_ref, bl_ref, out_ref, pooled, cnt):
    ib = pl.program_id(0)

    @pl.when(ib == 0)
    def _():
        pooled[...] = jnp.zeros_like(pooled)
        cnt[...] = jnp.zeros_like(cnt)

    dinv = _dinv_block(deg0_ref[...], deg1_ref[...])
    b = b_ref[...]
    r0 = jnp.maximum(dinv * a0_ref[...] + b[0:1, :], 0.0)
    r1 = jnp.maximum(dinv * a1_ref[...] + b[1:2, :], 0.0)
    h = jnp.concatenate([r0, r1], axis=1)
    bt = batch_ref[...][0]
    m = (lax.broadcasted_iota(jnp.int32, (G, RB), 0) == bt
         ).astype(jnp.float32)
    pooled[...] += jnp.dot(m, h, preferred_element_type=jnp.float32)
    cnt[...] += jnp.broadcast_to(jnp.sum(m, axis=1, keepdims=True), (G, 128))

    @pl.when(ib == pl.num_programs(0) - 1)
    def _():
        p = pooled[...] / jnp.maximum(cnt[:, :1], 1.0)
        out_ref[...] = (jnp.dot(p, wl_ref[...],
                                preferred_element_type=jnp.float32)
                        + bl_ref[...])


def _row_spec(cols):
    return pl.BlockSpec((RB, cols), lambda c, ib: (ib, 0))


def _hp1_call(x, w1, deg0, deg1):
    return pl.pallas_call(
        _hp1_body,
        grid=(NC, NBR),
        in_specs=[
            _row_spec(F_IN),
            pl.BlockSpec((F_IN, HALF), lambda c, ib: (0, c)),
            _row_spec(HALF), _row_spec(HALF),
        ],
        out_specs=pl.BlockSpec((RB, HALF), lambda c, ib: (c * NBR + ib, 0)),
        out_shape=jax.ShapeDtypeStruct((NC * NPAD, HALF), jnp.float32),
    )(x, w1, deg0, deg1)


def _hp2_call(a0, a1, deg0, deg1, b1r, w2):
    return pl.pallas_call(
        _hp2_body,
        grid=(NC, NBR),
        in_specs=[
            _row_spec(HALF), _row_spec(HALF),
            _row_spec(HALF), _row_spec(HALF),
            pl.BlockSpec((2, HALF), lambda c, ib: (0, 0)),
            pl.BlockSpec((HALF, HALF), lambda c, ib: (0, c)),
            pl.BlockSpec((HALF, HALF), lambda c, ib: (1, c)),
        ],
        out_specs=pl.BlockSpec((RB, HALF), lambda c, ib: (c * NBR + ib, 0)),
        out_shape=jax.ShapeDtypeStruct((NC * NPAD, HALF), jnp.float32),
    )(a0, a1, deg0, deg1, b1r, w2, w2)


def _final_call(a0, a1, deg0, deg1, b2r, batch3d, wl, bl2d):
    spec1 = pl.BlockSpec((RB, HALF), lambda ib: (ib, 0))
    spec16 = pl.BlockSpec((RB, HALF), lambda ib: (ib, 0))
    return pl.pallas_call(
        _final_body,
        grid=(NBR,),
        in_specs=[
            spec1, spec1, spec16, spec16,
            pl.BlockSpec((2, HALF), lambda ib: (0, 0)),
            pl.BlockSpec((1, 1, RB), lambda ib: (ib, 0, 0)),
            pl.BlockSpec((H, OUT), lambda ib: (0, 0)),
            pl.BlockSpec((1, OUT), lambda ib: (0, 0)),
        ],
        out_specs=pl.BlockSpec((G, OUT), lambda ib: (0, 0)),
        out_shape=jax.ShapeDtypeStruct((G, OUT), jnp.float32),
        scratch_shapes=[
            pltpu.VMEM((G, H), jnp.float32),
            pltpu.VMEM((G, 128), jnp.float32),
        ],
    )(a0, a1, deg0, deg1, b2r, batch3d, wl, bl2d)


def kernel(x, edge_index, batch, W1, b1, W2, b2, Wl, bl):
    src = edge_index[0].astype(jnp.int32)
    dst = edge_index[1].astype(jnp.int32)
    pad_e = EPAD - E
    srcp = jnp.concatenate([src, jnp.full((pad_e,), DUMMY, jnp.int32)])
    dstp = jnp.concatenate([dst, jnp.full((pad_e,), DUMMY, jnp.int32)])
    srcA = jnp.concatenate([srcp, srcp + NPAD]).reshape(-1, ECH)
    dst2d = dstp.reshape(-1, ECH)
    xp = jnp.zeros((NPAD, F_IN), jnp.float32).at[:N].set(x)
    batchp = jnp.full((NPAD,), G, jnp.int32).at[:N].set(batch.astype(jnp.int32))
    batch3d = batchp.reshape(NBR, 1, RB)
    zeros_h = jnp.zeros((ROWS_T, HALF), jnp.float32)
    ones_h = jnp.ones((ECH, HALF), jnp.float32)
    b1r = b1.reshape(2, HALF)
    b2r = b2.reshape(2, HALF)
    bl2d = bl.reshape(1, OUT)

    deg = _deg_kernel(dst2d, zeros_h, ones_h)
    deg0, deg1 = deg[:NPAD], deg[NPAD:]

    hp1 = _hp1_call(xp, W1, deg0, deg1)
    a1 = _prop_kernel(hp1, srcA, dst2d)
    hp2 = _hp2_call(a1[:NPAD], a1[NPAD:], deg0, deg1, b1r, W2)
    a2 = _prop_kernel(hp2, srcA, dst2d)
    return _final_call(a2[:NPAD], a2[NPAD:], deg0, deg1, b2r, batch3d, Wl,
                       bl2d)


# speedup vs baseline: 7.5649x; 1.0533x over previous
import functools

import jax
import jax.numpy as jnp
from jax import lax
from jax.experimental import pallas as pl
from jax.experimental.pallas import tpu as pltpu
from jax.experimental.pallas import tpu_sc as plsc

N = 10000
E = 160000
F_IN = 256
H = 256
OUT = 128
G = 64

NPAD = 10240
EPAD = 163840
DUMMY = N
NC, NS = 2, 16
HALF = 128
RB = 512
NBR = NPAD // RB
ECH = 128
ROWS_T = NPAD // NS
A_CH = EPAD // (NC * NS) // ECH
B_CH = EPAD // NS // ECH

_mesh = plsc.VectorSubcoreMesh(
    core_axis_name="c", subcore_axis_name="s", num_cores=NC, num_subcores=NS)


@functools.partial(
    pl.kernel,
    out_type=jax.ShapeDtypeStruct((NC * NPAD, HALF), jnp.float32),
    mesh=_mesh,
    scratch_types=[
        pltpu.VMEM_SHARED((NPAD, HALF), jnp.float32),
        pltpu.VMEM((4, ECH), jnp.int32),
        pltpu.VMEM((ECH, HALF), jnp.float32),
    ],
)
def _deg_kernel(dst_hbm, zeros_hbm, ones_hbm, out_hbm, acc, di, ones_v):
    c = lax.axis_index("c")
    s = lax.axis_index("s")
    w = c * NS + s
    pltpu.sync_copy(zeros_hbm, acc.at[pl.ds(s * ROWS_T, ROWS_T)])
    pltpu.sync_copy(ones_hbm, ones_v)
    plsc.subcore_barrier()
    drow = w * A_CH

    @pl.loop(0, A_CH, step=4)
    def _(g):
        pltpu.sync_copy(dst_hbm.at[pl.ds(drow + g, 4)], di)
        pltpu.sync_copy(ones_v, acc.at[di.at[0]], add=True)
        pltpu.sync_copy(ones_v, acc.at[di.at[1]], add=True)
        pltpu.sync_copy(ones_v, acc.at[di.at[2]], add=True)
        pltpu.sync_copy(ones_v, acc.at[di.at[3]], add=True)

    plsc.subcore_barrier()
    pltpu.sync_copy(acc.at[pl.ds(s * ROWS_T, ROWS_T)],
                    out_hbm.at[pl.ds(c * NPAD + s * ROWS_T, ROWS_T)])


@functools.partial(
    pl.kernel,
    out_type=jax.ShapeDtypeStruct((NC * NPAD, HALF), jnp.float32),
    mesh=_mesh,
    scratch_types=[
        pltpu.VMEM_SHARED((NPAD, HALF), jnp.float32),
        pltpu.VMEM((8, ECH), jnp.int32),
        pltpu.VMEM((8, ECH), jnp.int32),
        pltpu.VMEM((ECH, HALF), jnp.float32),
        pltpu.VMEM((ECH, HALF), jnp.float32),
        pltpu.SemaphoreType.DMA,
        pltpu.SemaphoreType.DMA,
    ],
)
def _prop_kernel(h_hbm, src_hbm, dst_hbm, out_hbm,
                 acc, si, di, r0, r1, sem0, sem1):
    c = lax.axis_index("c")
    s = lax.axis_index("s")
    pltpu.sync_copy(h_hbm.at[pl.ds(c * NPAD + s * ROWS_T, ROWS_T)],
                    acc.at[pl.ds(s * ROWS_T, ROWS_T)])
    plsc.subcore_barrier()

    srow = (c * NS + s) * B_CH
    drow = s * B_CH

    @pl.loop(0, B_CH, step=8)
    def _(g):
        pltpu.sync_copy(src_hbm.at[pl.ds(srow + g, 8)], si)
        pltpu.sync_copy(dst_hbm.at[pl.ds(drow + g, 8)], di)
        cp = pltpu.async_copy(h_hbm.at[si.at[0]], r0, sem0)
        cpn = pltpu.async_copy(h_hbm.at[si.at[1]], r1, sem1)
        for k in range(8):
            cp.wait()
            if k + 2 < 8:
                if k % 2 == 0:
                    pltpu.sync_copy(r0, acc.at[di.at[k]], add=True)
                    cp = cpn
                    cpn = pltpu.async_copy(h_hbm.at[si.at[k + 2]], r0, sem0)
                else:
                    pltpu.sync_copy(r1, acc.at[di.at[k]], add=True)
                    cp = cpn
                    cpn = pltpu.async_copy(h_hbm.at[si.at[k + 2]], r1, sem1)
            else:
                pltpu.sync_copy(r0 if k % 2 == 0 else r1,
                                acc.at[di.at[k]], add=True)
                cp = cpn

    plsc.subcore_barrier()
    pltpu.sync_copy(acc.at[pl.ds(s * ROWS_T, ROWS_T)],
                    out_hbm.at[pl.ds(c * NPAD + s * ROWS_T, ROWS_T)])


def _dinv_block(deg0, deg1):
    deg = deg0[:, :1] + deg1[:, :1] + 1.0
    return lax.rsqrt(jnp.maximum(deg, 1.0))


def _hp1_body(x_ref, w_ref, deg0_ref, deg1_ref, out_ref):
    dinv = _dinv_block(deg0_ref[...], deg1_ref[...])
    h = jnp.dot(x_ref[...], w_ref[...], preferred_element_type=jnp.float32)
    out_ref[...] = dinv * h


def _hp2_body(a0_ref, a1_ref, deg0_ref, deg1_ref, b_ref, w2a_ref, w2b_ref,
              out_ref):
    dinv = _dinv_block(deg0_ref[...], deg1_ref[...])
    b = b_ref[...]
    r0 = jnp.maximum(dinv * a0_ref[...] + b[0:1, :], 0.0)
    r1 = jnp.maximum(dinv * a1_ref[...] + b[1:2, :], 0.0)
    h = (jnp.dot(r0, w2a_ref[...], preferred_element_type=jnp.float32)
         + jnp.dot(r1, w2b_ref[...], preferred_element_type=jnp.float32))
    out_ref[...] = dinv * h


def _final_body(a0_ref, a1_ref, deg0_ref, deg1_ref, b_ref, batch_ref,
                wl_ref, bl_ref, out_ref, pooled, cnt):
    ib = pl.program_id(0)

    @pl.when(ib == 0)
    def _():
        pooled[...] = jnp.zeros_like(pooled)
        cnt[...] = jnp.zeros_like(cnt)

    dinv = _dinv_block(deg0_ref[...], deg1_ref[...])
    b = b_ref[...]
    r0 = jnp.maximum(dinv * a0_ref[...] + b[0:1, :], 0.0)
    r1 = jnp.maximum(dinv * a1_ref[...] + b[1:2, :], 0.0)
    h = jnp.concatenate([r0, r1], axis=1)
    bt = batch_ref[...][0]
    m = (lax.broadcasted_iota(jnp.int32, (G, RB), 0) == bt
         ).astype(jnp.float32)
    pooled[...] += jnp.dot(m, h, preferred_element_type=jnp.float32)
    cnt[...] += jnp.broadcast_to(jnp.sum(m, axis=1, keepdims=True), (G, 128))

    @pl.when(ib == pl.num_programs(0) - 1)
    def _():
        p = pooled[...] / jnp.maximum(cnt[:, :1], 1.0)
        out_ref[...] = (jnp.dot(p, wl_ref[...],
                                preferred_element_type=jnp.float32)
                        + bl_ref[...])


def _row_spec(cols):
    return pl.BlockSpec((RB, cols), lambda c, ib: (ib, 0))


def _hp1_call(x, w1, deg0, deg1):
    return pl.pallas_call(
        _hp1_body,
        grid=(NC, NBR),
        in_specs=[
            _row_spec(F_IN),
            pl.BlockSpec((F_IN, HALF), lambda c, ib: (0, c)),
            _row_spec(HALF), _row_spec(HALF),
        ],
        out_specs=pl.BlockSpec((RB, HALF), lambda c, ib: (c * NBR + ib, 0)),
        out_shape=jax.ShapeDtypeStruct((NC * NPAD, HALF), jnp.float32),
    )(x, w1, deg0, deg1)


def _hp2_call(a0, a1, deg0, deg1, b1r, w2):
    return pl.pallas_call(
        _hp2_body,
        grid=(NC, NBR),
        in_specs=[
            _row_spec(HALF), _row_spec(HALF),
            _row_spec(HALF), _row_spec(HALF),
            pl.BlockSpec((2, HALF), lambda c, ib: (0, 0)),
            pl.BlockSpec((HALF, HALF), lambda c, ib: (0, c)),
            pl.BlockSpec((HALF, HALF), lambda c, ib: (1, c)),
        ],
        out_specs=pl.BlockSpec((RB, HALF), lambda c, ib: (c * NBR + ib, 0)),
        out_shape=jax.ShapeDtypeStruct((NC * NPAD, HALF), jnp.float32),
    )(a0, a1, deg0, deg1, b1r, w2, w2)


def _final_call(a0, a1, deg0, deg1, b2r, batch3d, wl, bl2d):
    spec1 = pl.BlockSpec((RB, HALF), lambda ib: (ib, 0))
    spec16 = pl.BlockSpec((RB, HALF), lambda ib: (ib, 0))
    return pl.pallas_call(
        _final_body,
        grid=(NBR,),
        in_specs=[
            spec1, spec1, spec16, spec16,
            pl.BlockSpec((2, HALF), lambda ib: (0, 0)),
            pl.BlockSpec((1, 1, RB), lambda ib: (ib, 0, 0)),
            pl.BlockSpec((H, OUT), lambda ib: (0, 0)),
            pl.BlockSpec((1, OUT), lambda ib: (0, 0)),
        ],
        out_specs=pl.BlockSpec((G, OUT), lambda ib: (0, 0)),
        out_shape=jax.ShapeDtypeStruct((G, OUT), jnp.float32),
        scratch_shapes=[
            pltpu.VMEM((G, H), jnp.float32),
            pltpu.VMEM((G, 128), jnp.float32),
        ],
    )(a0, a1, deg0, deg1, b2r, batch3d, wl, bl2d)


def kernel(x, edge_index, batch, W1, b1, W2, b2, Wl, bl):
    src = edge_index[0].astype(jnp.int32)
    dst = edge_index[1].astype(jnp.int32)
    pad_e = EPAD - E
    srcp = jnp.concatenate([src, jnp.full((pad_e,), DUMMY, jnp.int32)])
    dstp = jnp.concatenate([dst, jnp.full((pad_e,), DUMMY, jnp.int32)])
    srcA = jnp.concatenate([srcp, srcp + NPAD]).reshape(-1, ECH)
    dst2d = dstp.reshape(-1, ECH)
    xp = jnp.zeros((NPAD, F_IN), jnp.float32).at[:N].set(x)
    batchp = jnp.full((NPAD,), G, jnp.int32).at[:N].set(batch.astype(jnp.int32))
    batch3d = batchp.reshape(NBR, 1, RB)
    zeros_h = jnp.zeros((ROWS_T, HALF), jnp.float32)
    ones_h = jnp.ones((ECH, HALF), jnp.float32)
    b1r = b1.reshape(2, HALF)
    b2r = b2.reshape(2, HALF)
    bl2d = bl.reshape(1, OUT)

    deg = _deg_kernel(dst2d, zeros_h, ones_h)
    deg0, deg1 = deg[:NPAD], deg[NPAD:]

    hp1 = _hp1_call(xp, W1, deg0, deg1)
    a1 = _prop_kernel(hp1, srcA, dst2d)
    hp2 = _hp2_call(a1[:NPAD], a1[NPAD:], deg0, deg1, b1r, W2)
    a2 = _prop_kernel(hp2, srcA, dst2d)
    return _final_call(a2[:NPAD], a2[NPAD:], deg0, deg1, b2r, batch3d, Wl,
                       bl2d)
